# baseline (speedup 1.0000x reference)
import functools

import jax
import jax.numpy as jnp
from jax import lax
from jax.experimental import pallas as pl
from jax.experimental.pallas import tpu as pltpu
from jax.experimental.pallas import tpu_sc as plsc

NUM_CLASSES = 1000000
COND_SIZE = 64
BATCH = 16384
OUT_STRIDE = 128

NUM_CORES = 2
NUM_SUBCORES = 16
NUM_WORKERS = NUM_CORES * NUM_SUBCORES

CHUNK_L = 512
N_CHUNKS = 1953
V_FULL = N_CHUNKS * CHUNK_L
NCH_BASE = 61

WL_CAP = 704
CH_CAP = 256
B_CAP = 96
BL_LEN = 4 * B_CAP + 32
ROWS_CAP = 640
SENTINEL = 0x7FFF0000
BIG = 1 << 30


def _make_scan():
    mesh = plsc.VectorSubcoreMesh(core_axis_name="c", subcore_axis_name="s")

    @functools.partial(
        pl.kernel,
        mesh=mesh,
        out_type=jax.ShapeDtypeStruct((BATCH * OUT_STRIDE,), jnp.float32),
        scratch_types=[
            pltpu.VMEM((BATCH,), jnp.int32),
            pltpu.VMEM((WL_CAP,), jnp.int32),
            pltpu.VMEM((WL_CAP,), jnp.int32),
            pltpu.VMEM((CH_CAP,), jnp.int32),
            pltpu.VMEM((CH_CAP,), jnp.int32),
            pltpu.VMEM((BL_LEN,), jnp.int32),
            pltpu.VMEM((BL_LEN,), jnp.int32),
            pltpu.VMEM((COND_SIZE, 128), jnp.float32),
            pltpu.VMEM((COND_SIZE, 128), jnp.float32),
            pltpu.VMEM((COND_SIZE, 128), jnp.float32),
            pltpu.VMEM((COND_SIZE, 128), jnp.float32),
            pltpu.VMEM((COND_SIZE, 128), jnp.float32),
            pltpu.VMEM((COND_SIZE, 128), jnp.float32),
            pltpu.VMEM((COND_SIZE, 128), jnp.float32),
            pltpu.VMEM((COND_SIZE, 128), jnp.float32),
            pltpu.VMEM((ROWS_CAP * COND_SIZE,), jnp.float32),
            pltpu.VMEM((WL_CAP,), jnp.int32),
            pltpu.SemaphoreType.DMA,
            pltpu.SemaphoreType.DMA,
            pltpu.SemaphoreType.DMA,
        ],
        compiler_params=pltpu.CompilerParams(
            use_tc_tiling_on_sc=True, needs_layout_passes=False
        ),
    )
    def scan_kernel(labels_hbm, table_hbm, out_hbm,
                    idx_all, wl_lab, wl_pos, ch_lab, ch_pos,
                    bl_a, bl_b,
                    a0, a1, a2, a3, b0, b1, b2, b3,
                    rows, pos_stage, sem_a, sem_b, sem_o):
        bufs_a = (a0, a1, a2, a3)
        bufs_b = (b0, b1, b2, b3)
        iota = lax.iota(jnp.int32, 16)

        wid = lax.axis_index("s") * NUM_CORES + lax.axis_index("c")
        nch = jnp.where(wid == NUM_WORKERS - 1, NCH_BASE + 1, NCH_BASE)
        lo = wid * (NCH_BASE * CHUNK_L)
        hi = lo + nch * CHUNK_L
        tail_off = jnp.minimum(lo, 0) + V_FULL

        def fetch(bufs, sem, off):
            for t in range(4):
                o = pl.multiple_of(off + 128 * t, 128)
                pltpu.async_copy(
                    table_hbm.at[:, pl.ds(o, 128)], bufs[t], sem
                )

        def waitk(bufs, sem, k):
            def w(i, _):
                pltpu.make_async_copy(
                    table_hbm.at[:, pl.ds(0, 128)], bufs[0], sem
                ).wait()
                return 0

            lax.fori_loop(0, k, w, 0)

        fetch(bufs_a, sem_a, lo)
        fetch(bufs_b, sem_b, lo + CHUNK_L)

        pltpu.sync_copy(labels_hbm, idx_all)

        sent = jnp.full((16,), SENTINEL, jnp.int32)

        def init_wl(v, _):
            wl_lab[pl.ds(v * 16, 16)] = sent
            return 0

        lax.fori_loop(0, WL_CAP // 16, init_wl, 0)

        def phase_a(v, n):
            lab = idx_all[pl.ds(v * 16, 16)]
            span = (hi - lo).astype(jnp.uint32)
            m = ((lab - lo).astype(jnp.uint32) < span) | (lab >= V_FULL)
            plsc.store_compressed(wl_lab.at[pl.ds(n, 16)], lab, mask=m)
            plsc.store_compressed(
                wl_pos.at[pl.ds(n, 16)], v * 16 + iota, mask=m
            )
            return n + plsc.all_reduce_population_count(m)[0]

        lax.fori_loop(0, BATCH // 16, phase_a, jnp.int32(0))

        def fbase_of(g):
            return jnp.where(g < nch, lo + g * CHUNK_L, jnp.int32(BIG))

        def filt(bl, fbase):
            def s1(v, c):
                lab = wl_lab[pl.ds(v * 16, 16)]
                pos = wl_pos[pl.ds(v * 16, 16)]
                m = (lab - fbase).astype(jnp.uint32) < jnp.uint32(CHUNK_L)
                plsc.store_compressed(ch_lab.at[pl.ds(c, 16)], lab, mask=m)
                plsc.store_compressed(ch_pos.at[pl.ds(c, 16)], pos, mask=m)
                return c + plsc.all_reduce_population_count(m)[0]

            ch_n = lax.fori_loop(0, WL_CAP // 16, s1, jnp.int32(0))

            cnts = []
            for t in range(4):
                base_t = fbase + 128 * t

                def s2(v, bn):
                    idxv = v * 16 + iota
                    lab = ch_lab[pl.ds(v * 16, 16)]
                    pos = ch_pos[pl.ds(v * 16, 16)]
                    lane = lab - base_t
                    m = (idxv < ch_n) & (lane.astype(jnp.uint32) < jnp.uint32(128))
                    plsc.store_compressed(
                        bl.at[pl.ds(t * B_CAP + bn, 16)], pos * 128 + lane,
                        mask=m,
                    )
                    return bn + plsc.all_reduce_population_count(m)[0]

                cnts.append(lax.fori_loop(0, 4, s2, jnp.int32(0)))
            return tuple(cnts)

        def start_cond(bufs, sem, g, cnts):
            real = jnp.minimum(g, nch - 1)
            off = lo + real * CHUNK_L
            for t in range(4):
                @pl.when(cnts[t] > 0)
                def _():
                    o = pl.multiple_of(off + 128 * t, 128)
                    pltpu.async_copy(
                        table_hbm.at[:, pl.ds(o, 128)], bufs[t], sem
                    )

        def waitn(bufs, sem, cnts):
            waitk(bufs, sem, sum((c > 0).astype(jnp.int32) for c in cnts))

        def extract_slot(bufs, bl, cnts, h):
            for t in range(4):
                buf = bufs[t]

                def extract(e, hh):
                    w = bl[pl.ds(t * B_CAP + e, 16)][0]
                    r = w & 127
                    pv = w >> 7
                    rvec = jnp.full((16,), r, jnp.int32)
                    for kk in range(4):
                        vals = plsc.load_gather(buf, [iota + 16 * kk, rvec])
                        rows[pl.ds(hh * COND_SIZE + 16 * kk, 16)] = vals
                    plsc.store_scatter(
                        pos_stage,
                        [jnp.full((16,), hh, jnp.int32)],
                        jnp.full((16,), pv, jnp.int32),
                    )
                    return hh + 1

                h = lax.fori_loop(0, cnts[t], extract, h)
            return h

        c_a = filt(bl_a, fbase_of(jnp.int32(0)))
        waitk(bufs_a, sem_a, 4)
        h = extract_slot(bufs_a, bl_a, c_a, jnp.int32(0))
        c_a2 = filt(bl_a, fbase_of(jnp.int32(2)))
        start_cond(bufs_a, sem_a, jnp.int32(2), c_a2)
        c_b = filt(bl_b, fbase_of(jnp.int32(1)))
        waitk(bufs_b, sem_b, 4)
        h = extract_slot(bufs_b, bl_b, c_b, h)

        def pair(i, carry):
            h, c_a2 = carry[0], carry[1:5]
            g_b = 2 * i + 3
            c_b2 = filt(bl_b, fbase_of(g_b))
            start_cond(bufs_b, sem_b, g_b, c_b2)
            waitn(bufs_a, sem_a, c_a2)
            h = extract_slot(bufs_a, bl_a, c_a2, h)
            c_a3 = filt(bl_a, fbase_of(g_b + 1))
            start_cond(bufs_a, sem_a, g_b + 1, c_a3)
            waitn(bufs_b, sem_b, c_b2)
            h = extract_slot(bufs_b, bl_b, c_b2, h)
            return (h,) + c_a3

        carry = lax.fori_loop(0, 30, pair, (h,) + c_a2)
        h = carry[0]

        c_t = filt(bl_a, tail_off)
        pltpu.async_copy(
            table_hbm.at[:, pl.ds(pl.multiple_of(tail_off, 128), 128)],
            a0,
            sem_a,
        )
        waitk(bufs_a, sem_a, 1)
        h = extract_slot(bufs_a, bl_a, c_t, h)

        def fire(e, _):
            pv = pos_stage[pl.ds(e, 16)][0]
            pltpu.async_copy(
                rows.at[pl.ds(pl.multiple_of(e * COND_SIZE, 8), COND_SIZE)],
                out_hbm.at[
                    pl.ds(pl.multiple_of(pv * OUT_STRIDE, 8), COND_SIZE)
                ],
                sem_o,
            )
            return 0

        lax.fori_loop(0, h, fire, 0)

        def drain(e, _):
            pltpu.make_async_copy(
                rows.at[pl.ds(0, COND_SIZE)],
                out_hbm.at[pl.ds(0, COND_SIZE)],
                sem_o,
            ).wait()
            return 0

        lax.fori_loop(0, h, drain, 0)

    return scan_kernel


_scan = _make_scan()


def kernel(labels, embedding_table):
    labels = labels.astype(jnp.int32)
    table_t = embedding_table.T
    flat = _scan(labels, table_t)
    return flat.reshape(BATCH, OUT_STRIDE)[:, :COND_SIZE]


# speedup vs baseline: 1.0632x; 1.0632x over previous
import functools

import jax
import jax.numpy as jnp
from jax import lax
from jax.experimental import pallas as pl
from jax.experimental.pallas import tpu as pltpu
from jax.experimental.pallas import tpu_sc as plsc

NUM_CLASSES = 1000000
COND_SIZE = 64
BATCH = 16384
OUT_STRIDE = 128

NUM_CORES = 2
NUM_SUBCORES = 16
NUM_WORKERS = NUM_CORES * NUM_SUBCORES

CHUNK_L = 512
N_CHUNKS = 1953
V_FULL = N_CHUNKS * CHUNK_L
NCH_BASE = 61

WL_CAP = 704
CH_CAP = 256
B_CAP = 96
BL_LEN = 4 * B_CAP + 32
ROWS_CAP = 640
SENTINEL = 0x7FFF0000
BIG = 1 << 30


def _make_scan():
    mesh = plsc.VectorSubcoreMesh(core_axis_name="c", subcore_axis_name="s")

    @functools.partial(
        pl.kernel,
        mesh=mesh,
        out_type=jax.ShapeDtypeStruct((BATCH * OUT_STRIDE,), jnp.float32),
        scratch_types=[
            pltpu.VMEM((BATCH,), jnp.int32),
            pltpu.VMEM((WL_CAP,), jnp.int32),
            pltpu.VMEM((WL_CAP,), jnp.int32),
            pltpu.VMEM((CH_CAP,), jnp.int32),
            pltpu.VMEM((CH_CAP,), jnp.int32),
            pltpu.VMEM((BL_LEN,), jnp.int32),
            pltpu.VMEM((BL_LEN,), jnp.int32),
            pltpu.VMEM((BL_LEN,), jnp.int32),
            pltpu.VMEM((BL_LEN,), jnp.int32),
            pltpu.VMEM((COND_SIZE, 128), jnp.float32),
            pltpu.VMEM((COND_SIZE, 128), jnp.float32),
            pltpu.VMEM((COND_SIZE, 128), jnp.float32),
            pltpu.VMEM((COND_SIZE, 128), jnp.float32),
            pltpu.VMEM((COND_SIZE, 128), jnp.float32),
            pltpu.VMEM((COND_SIZE, 128), jnp.float32),
            pltpu.VMEM((COND_SIZE, 128), jnp.float32),
            pltpu.VMEM((COND_SIZE, 128), jnp.float32),
            pltpu.VMEM((ROWS_CAP * COND_SIZE,), jnp.float32),
            pltpu.SemaphoreType.DMA,
            pltpu.SemaphoreType.DMA,
            pltpu.SemaphoreType.DMA,
        ],
        compiler_params=pltpu.CompilerParams(
            use_tc_tiling_on_sc=True, needs_layout_passes=False
        ),
    )
    def scan_kernel(labels_hbm, table_hbm, out_hbm,
                    idx_all, wl_lab, wl_pos, ch_lab, ch_pos,
                    bl_a, bp_a, bl_b, bp_b,
                    a0, a1, a2, a3, b0, b1, b2, b3,
                    rows, sem_a, sem_b, sem_o):
        bufs_a = (a0, a1, a2, a3)
        bufs_b = (b0, b1, b2, b3)
        iota = lax.iota(jnp.int32, 16)

        wid = lax.axis_index("s") * NUM_CORES + lax.axis_index("c")
        nch = jnp.where(wid == NUM_WORKERS - 1, NCH_BASE + 1, NCH_BASE)
        lo = wid * (NCH_BASE * CHUNK_L)
        hi = lo + nch * CHUNK_L
        tail_off = jnp.minimum(lo, 0) + V_FULL

        def fetch(bufs, sem, off):
            for t in range(4):
                o = pl.multiple_of(off + 128 * t, 128)
                pltpu.async_copy(
                    table_hbm.at[:, pl.ds(o, 128)], bufs[t], sem
                )

        def waitk(bufs, sem, k):
            def w(i, _):
                pltpu.make_async_copy(
                    table_hbm.at[:, pl.ds(0, 128)], bufs[0], sem
                ).wait()
                return 0

            lax.fori_loop(0, k, w, 0)

        fetch(bufs_a, sem_a, lo)
        fetch(bufs_b, sem_b, lo + CHUNK_L)

        pltpu.sync_copy(labels_hbm, idx_all)

        sent = jnp.full((16,), SENTINEL, jnp.int32)

        def init_wl(v, _):
            wl_lab[pl.ds(v * 16, 16)] = sent
            return 0

        lax.fori_loop(0, WL_CAP // 16, init_wl, 0)

        def phase_a(v, n):
            lab = idx_all[pl.ds(v * 16, 16)]
            m = ((lab >= lo) & (lab < hi)) | (lab >= V_FULL)
            plsc.store_compressed(wl_lab.at[pl.ds(n, 16)], lab, mask=m)
            plsc.store_compressed(
                wl_pos.at[pl.ds(n, 16)], v * 16 + iota, mask=m
            )
            return n + plsc.all_reduce_population_count(m)[0]

        lax.fori_loop(0, BATCH // 16, phase_a, jnp.int32(0))

        def fbase_of(g):
            return jnp.where(g < nch, lo + g * CHUNK_L, jnp.int32(BIG))

        def filt(bl, bp, fbase):
            def s1(v, c):
                lab = wl_lab[pl.ds(v * 16, 16)]
                pos = wl_pos[pl.ds(v * 16, 16)]
                m = (lab >= fbase) & (lab < fbase + CHUNK_L)
                plsc.store_compressed(ch_lab.at[pl.ds(c, 16)], lab, mask=m)
                plsc.store_compressed(ch_pos.at[pl.ds(c, 16)], pos, mask=m)
                return c + plsc.all_reduce_population_count(m)[0]

            ch_n = lax.fori_loop(0, WL_CAP // 16, s1, jnp.int32(0))

            cnts = []
            for t in range(4):
                base_t = fbase + 128 * t

                def s2(v, bn):
                    idxv = v * 16 + iota
                    lab = ch_lab[pl.ds(v * 16, 16)]
                    pos = ch_pos[pl.ds(v * 16, 16)]
                    m = (idxv < ch_n) & (lab >= base_t) & (lab < base_t + 128)
                    plsc.store_compressed(
                        bl.at[pl.ds(t * B_CAP + bn, 16)], lab - base_t, mask=m
                    )
                    plsc.store_compressed(
                        bp.at[pl.ds(t * B_CAP + bn, 16)], pos, mask=m
                    )
                    return bn + plsc.all_reduce_population_count(m)[0]

                cnts.append(lax.fori_loop(0, 4, s2, jnp.int32(0)))
            return tuple(cnts)

        def start_cond(bufs, sem, g, cnts):
            real = jnp.minimum(g, nch - 1)
            off = lo + real * CHUNK_L
            for t in range(4):
                @pl.when(cnts[t] > 0)
                def _():
                    o = pl.multiple_of(off + 128 * t, 128)
                    pltpu.async_copy(
                        table_hbm.at[:, pl.ds(o, 128)], bufs[t], sem
                    )

        def waitn(bufs, sem, cnts):
            waitk(bufs, sem, sum((c > 0).astype(jnp.int32) for c in cnts))

        def extract_slot(bufs, bl, bp, cnts, h):
            for t in range(4):
                buf = bufs[t]

                def extract(e, hh):
                    r = bl[pl.ds(t * B_CAP + e, 16)][0]
                    pv = bp[pl.ds(t * B_CAP + e, 16)][0]
                    rvec = jnp.full((16,), r, jnp.int32)
                    for kk in range(4):
                        vals = plsc.load_gather(buf, [iota + 16 * kk, rvec])
                        rows[pl.ds(hh * COND_SIZE + 16 * kk, 16)] = vals
                    pltpu.async_copy(
                        rows.at[
                            pl.ds(pl.multiple_of(hh * COND_SIZE, 8), COND_SIZE)
                        ],
                        out_hbm.at[
                            pl.ds(pl.multiple_of(pv * OUT_STRIDE, 8), COND_SIZE)
                        ],
                        sem_o,
                    )
                    return hh + 1

                h = lax.fori_loop(0, cnts[t], extract, h)
            return h

        c_a = filt(bl_a, bp_a, fbase_of(jnp.int32(0)))
        waitk(bufs_a, sem_a, 4)
        h = extract_slot(bufs_a, bl_a, bp_a, c_a, jnp.int32(0))
        c_a2 = filt(bl_a, bp_a, fbase_of(jnp.int32(2)))
        start_cond(bufs_a, sem_a, jnp.int32(2), c_a2)
        c_b = filt(bl_b, bp_b, fbase_of(jnp.int32(1)))
        waitk(bufs_b, sem_b, 4)
        h = extract_slot(bufs_b, bl_b, bp_b, c_b, h)

        def pair(i, carry):
            h, c_a2 = carry[0], carry[1:5]
            g_b = 2 * i + 3
            c_b2 = filt(bl_b, bp_b, fbase_of(g_b))
            start_cond(bufs_b, sem_b, g_b, c_b2)
            waitn(bufs_a, sem_a, c_a2)
            h = extract_slot(bufs_a, bl_a, bp_a, c_a2, h)
            c_a3 = filt(bl_a, bp_a, fbase_of(g_b + 1))
            start_cond(bufs_a, sem_a, g_b + 1, c_a3)
            waitn(bufs_b, sem_b, c_b2)
            h = extract_slot(bufs_b, bl_b, bp_b, c_b2, h)
            return (h,) + c_a3

        carry = lax.fori_loop(0, 30, pair, (h,) + c_a2)
        h = carry[0]

        c_t = filt(bl_a, bp_a, tail_off)
        pltpu.async_copy(
            table_hbm.at[:, pl.ds(pl.multiple_of(tail_off, 128), 128)],
            a0,
            sem_a,
        )
        waitk(bufs_a, sem_a, 1)
        h = extract_slot(bufs_a, bl_a, bp_a, c_t, h)

        def drain(e, _):
            pltpu.make_async_copy(
                rows.at[pl.ds(0, COND_SIZE)],
                out_hbm.at[pl.ds(0, COND_SIZE)],
                sem_o,
            ).wait()
            return 0

        lax.fori_loop(0, h, drain, 0)

    return scan_kernel


_scan = _make_scan()


def kernel(labels, embedding_table):
    labels = labels.astype(jnp.int32)
    table_t = embedding_table.T
    flat = _scan(labels, table_t)
    return flat.reshape(BATCH, OUT_STRIDE)[:, :COND_SIZE]


# speedup vs baseline: 1.0695x; 1.0060x over previous
import functools

import jax
import jax.numpy as jnp
from jax import lax
from jax.experimental import pallas as pl
from jax.experimental.pallas import tpu as pltpu
from jax.experimental.pallas import tpu_sc as plsc

NUM_CLASSES = 1000000
COND_SIZE = 64
BATCH = 16384
OUT_STRIDE = 128

NUM_CORES = 2
NUM_SUBCORES = 16
NUM_WORKERS = NUM_CORES * NUM_SUBCORES

CHUNK_L = 512
N_CHUNKS = 1953
V_FULL = N_CHUNKS * CHUNK_L
NCH_BASE = 61

WL_CAP = 704
CH_CAP = 256
B_CAP = 96
BL_LEN = 4 * B_CAP + 32
ROWS_CAP = 640
SENTINEL = 0x7FFF0000
BIG = 1 << 30


def _make_scan():
    mesh = plsc.VectorSubcoreMesh(core_axis_name="c", subcore_axis_name="s")

    @functools.partial(
        pl.kernel,
        mesh=mesh,
        out_type=jax.ShapeDtypeStruct((BATCH * OUT_STRIDE,), jnp.float32),
        scratch_types=[
            pltpu.VMEM((BATCH,), jnp.int32),
            pltpu.VMEM((WL_CAP,), jnp.int32),
            pltpu.VMEM((WL_CAP,), jnp.int32),
            pltpu.VMEM((CH_CAP,), jnp.int32),
            pltpu.VMEM((CH_CAP,), jnp.int32),
            pltpu.VMEM((BL_LEN,), jnp.int32),
            pltpu.VMEM((BL_LEN,), jnp.int32),
            pltpu.VMEM((BL_LEN,), jnp.int32),
            pltpu.VMEM((BL_LEN,), jnp.int32),
            pltpu.VMEM((COND_SIZE, 128), jnp.float32),
            pltpu.VMEM((COND_SIZE, 128), jnp.float32),
            pltpu.VMEM((COND_SIZE, 128), jnp.float32),
            pltpu.VMEM((COND_SIZE, 128), jnp.float32),
            pltpu.VMEM((COND_SIZE, 128), jnp.float32),
            pltpu.VMEM((COND_SIZE, 128), jnp.float32),
            pltpu.VMEM((COND_SIZE, 128), jnp.float32),
            pltpu.VMEM((COND_SIZE, 128), jnp.float32),
            pltpu.VMEM((ROWS_CAP * COND_SIZE,), jnp.float32),
            pltpu.SemaphoreType.DMA,
            pltpu.SemaphoreType.DMA,
            pltpu.SemaphoreType.DMA,
        ],
        compiler_params=pltpu.CompilerParams(
            use_tc_tiling_on_sc=True, needs_layout_passes=False
        ),
    )
    def scan_kernel(labels_hbm, table_hbm, out_hbm,
                    idx_all, wl_lab, wl_pos, ch_lab, ch_pos,
                    bl_a, bp_a, bl_b, bp_b,
                    a0, a1, a2, a3, b0, b1, b2, b3,
                    rows, sem_a, sem_b, sem_o):
        bufs_a = (a0, a1, a2, a3)
        bufs_b = (b0, b1, b2, b3)
        iota = lax.iota(jnp.int32, 16)

        wid = lax.axis_index("s") * NUM_CORES + lax.axis_index("c")
        nch = jnp.where(wid == NUM_WORKERS - 1, NCH_BASE + 1, NCH_BASE)
        lo = wid * (NCH_BASE * CHUNK_L)
        hi = lo + nch * CHUNK_L
        tail_off = jnp.minimum(lo, 0) + V_FULL

        def fetch(bufs, sem, off):
            for t in range(4):
                o = pl.multiple_of(off + 128 * t, 128)
                pltpu.async_copy(
                    table_hbm.at[:, pl.ds(o, 128)], bufs[t], sem
                )

        def waitk(bufs, sem, k):
            def w(i, _):
                pltpu.make_async_copy(
                    table_hbm.at[:, pl.ds(0, 128)], bufs[0], sem
                ).wait()
                return 0

            lax.fori_loop(0, k, w, 0)

        fetch(bufs_a, sem_a, lo)
        fetch(bufs_b, sem_b, lo + CHUNK_L)

        pltpu.sync_copy(labels_hbm, idx_all)

        sent = jnp.full((16,), SENTINEL, jnp.int32)

        def init_wl(v, _):
            wl_lab[pl.ds(v * 16, 16)] = sent
            return 0

        lax.fori_loop(0, WL_CAP // 16, init_wl, 0, unroll=4)

        def phase_a(v, n):
            lab = idx_all[pl.ds(v * 16, 16)]
            m = ((lab >= lo) & (lab < hi)) | (lab >= V_FULL)
            plsc.store_compressed(wl_lab.at[pl.ds(n, 16)], lab, mask=m)
            plsc.store_compressed(
                wl_pos.at[pl.ds(n, 16)], v * 16 + iota, mask=m
            )
            return n + plsc.all_reduce_population_count(m)[0]

        lax.fori_loop(0, BATCH // 16, phase_a, jnp.int32(0), unroll=4)

        def fbase_of(g):
            return jnp.where(g < nch, lo + g * CHUNK_L, jnp.int32(BIG))

        def filt(bl, bp, fbase):
            def s1(v, c):
                lab = wl_lab[pl.ds(v * 16, 16)]
                pos = wl_pos[pl.ds(v * 16, 16)]
                m = (lab >= fbase) & (lab < fbase + CHUNK_L)
                plsc.store_compressed(ch_lab.at[pl.ds(c, 16)], lab, mask=m)
                plsc.store_compressed(ch_pos.at[pl.ds(c, 16)], pos, mask=m)
                return c + plsc.all_reduce_population_count(m)[0]

            ch_n = lax.fori_loop(0, WL_CAP // 16, s1, jnp.int32(0))

            cnts = []
            for t in range(4):
                base_t = fbase + 128 * t

                def s2(v, bn):
                    idxv = v * 16 + iota
                    lab = ch_lab[pl.ds(v * 16, 16)]
                    pos = ch_pos[pl.ds(v * 16, 16)]
                    m = (idxv < ch_n) & (lab >= base_t) & (lab < base_t + 128)
                    plsc.store_compressed(
                        bl.at[pl.ds(t * B_CAP + bn, 16)], lab - base_t, mask=m
                    )
                    plsc.store_compressed(
                        bp.at[pl.ds(t * B_CAP + bn, 16)], pos, mask=m
                    )
                    return bn + plsc.all_reduce_population_count(m)[0]

                cnts.append(lax.fori_loop(0, 4, s2, jnp.int32(0)))
            return tuple(cnts)

        def start_cond(bufs, sem, g, cnts):
            real = jnp.minimum(g, nch - 1)
            off = lo + real * CHUNK_L
            for t in range(4):
                @pl.when(cnts[t] > 0)
                def _():
                    o = pl.multiple_of(off + 128 * t, 128)
                    pltpu.async_copy(
                        table_hbm.at[:, pl.ds(o, 128)], bufs[t], sem
                    )

        def waitn(bufs, sem, cnts):
            waitk(bufs, sem, sum((c > 0).astype(jnp.int32) for c in cnts))

        def extract_slot(bufs, bl, bp, cnts, h):
            for t in range(4):
                buf = bufs[t]

                def extract(e, hh):
                    r = bl[pl.ds(t * B_CAP + e, 16)][0]
                    pv = bp[pl.ds(t * B_CAP + e, 16)][0]
                    rvec = jnp.full((16,), r, jnp.int32)
                    for kk in range(4):
                        vals = plsc.load_gather(buf, [iota + 16 * kk, rvec])
                        rows[pl.ds(hh * COND_SIZE + 16 * kk, 16)] = vals
                    pltpu.async_copy(
                        rows.at[
                            pl.ds(pl.multiple_of(hh * COND_SIZE, 8), COND_SIZE)
                        ],
                        out_hbm.at[
                            pl.ds(pl.multiple_of(pv * OUT_STRIDE, 8), COND_SIZE)
                        ],
                        sem_o,
                    )
                    return hh + 1

                h = lax.fori_loop(0, cnts[t], extract, h)
            return h

        c_a = filt(bl_a, bp_a, fbase_of(jnp.int32(0)))
        waitk(bufs_a, sem_a, 4)
        h = extract_slot(bufs_a, bl_a, bp_a, c_a, jnp.int32(0))
        c_a2 = filt(bl_a, bp_a, fbase_of(jnp.int32(2)))
        start_cond(bufs_a, sem_a, jnp.int32(2), c_a2)
        c_b = filt(bl_b, bp_b, fbase_of(jnp.int32(1)))
        waitk(bufs_b, sem_b, 4)
        h = extract_slot(bufs_b, bl_b, bp_b, c_b, h)

        def pair(i, carry):
            h, c_a2 = carry[0], carry[1:5]
            g_b = 2 * i + 3
            c_b2 = filt(bl_b, bp_b, fbase_of(g_b))
            start_cond(bufs_b, sem_b, g_b, c_b2)
            waitn(bufs_a, sem_a, c_a2)
            h = extract_slot(bufs_a, bl_a, bp_a, c_a2, h)
            c_a3 = filt(bl_a, bp_a, fbase_of(g_b + 1))
            start_cond(bufs_a, sem_a, g_b + 1, c_a3)
            waitn(bufs_b, sem_b, c_b2)
            h = extract_slot(bufs_b, bl_b, bp_b, c_b2, h)
            return (h,) + c_a3

        carry = lax.fori_loop(0, 30, pair, (h,) + c_a2)
        h = carry[0]

        c_t = filt(bl_a, bp_a, tail_off)
        pltpu.async_copy(
            table_hbm.at[:, pl.ds(pl.multiple_of(tail_off, 128), 128)],
            a0,
            sem_a,
        )
        waitk(bufs_a, sem_a, 1)
        h = extract_slot(bufs_a, bl_a, bp_a, c_t, h)

        def drain(e, _):
            pltpu.make_async_copy(
                rows.at[pl.ds(0, COND_SIZE)],
                out_hbm.at[pl.ds(0, COND_SIZE)],
                sem_o,
            ).wait()
            return 0

        lax.fori_loop(0, h, drain, 0)

    return scan_kernel


_scan = _make_scan()


def kernel(labels, embedding_table):
    labels = labels.astype(jnp.int32)
    table_t = embedding_table.T
    flat = _scan(labels, table_t)
    return flat.reshape(BATCH, OUT_STRIDE)[:, :COND_SIZE]
